# Initial kernel scaffold; baseline (speedup 1.0000x reference)
import jax
import jax.numpy as jnp
from jax.experimental import pallas as pl


def kernel(x, edge_index, W, b):
    raise NotImplementedError("write your pallas kernel here")


# speedup vs baseline: 7.3914x; 7.3914x over previous
import functools

import jax
import jax.numpy as jnp
from jax import lax
from jax.experimental import pallas as pl
from jax.experimental.pallas import tpu as pltpu
from jax.experimental.pallas import tpu_sc as plsc

N = 10000
E = 320000
D = 128

NC = 2
NS = 16
NW = NC * NS

E_PER_W = E // NW
CHUNK = 80
NITER = E_PER_W // CHUNK
N_PAD = 10112
STRIPE = N_PAD // NS


_sc_mesh = plsc.VectorSubcoreMesh(core_axis_name="c", subcore_axis_name="s")


@functools.partial(
    pl.kernel,
    out_type=jax.ShapeDtypeStruct((NC, N_PAD, D), jnp.float32),
    mesh=_sc_mesh,
    scratch_types=[
        pltpu.VMEM((NITER, CHUNK), jnp.int32),
        pltpu.VMEM((NITER, CHUNK), jnp.int32),
        pltpu.VMEM((CHUNK, D), jnp.float32),
        pltpu.VMEM_SHARED((N_PAD, D), jnp.float32),
        pltpu.SemaphoreType.DMA,
    ],
)
def _sc_aggregate(x_hbm, src_hbm, dst_hbm, zeros_hbm, out_hbm,
                  src_v, dst_v, rows_v, agg_sh, sem):
    cid = lax.axis_index("c")
    sid = lax.axis_index("s")
    wid = sid * NC + cid

    pltpu.sync_copy(zeros_hbm, agg_sh.at[pl.ds(sid * STRIPE, STRIPE)])
    pltpu.sync_copy(src_hbm.at[wid], src_v)
    pltpu.sync_copy(dst_hbm.at[wid], dst_v)
    plsc.subcore_barrier()

    def body(i, _):
        pltpu.async_copy(x_hbm.at[src_v.at[i]], rows_v, sem).wait()
        pltpu.sync_copy(rows_v, agg_sh.at[dst_v.at[i]], add=True)
        return ()

    lax.fori_loop(0, NITER, body, (), unroll=False)

    plsc.subcore_barrier()
    pltpu.sync_copy(
        agg_sh.at[pl.ds(sid * STRIPE, STRIPE)],
        out_hbm.at[cid, pl.ds(sid * STRIPE, STRIPE)],
    )


_BLK = 632


def _tc_linear_body(agg_ref, w_ref, b_ref, o_ref):
    a = agg_ref[0] + agg_ref[1]
    h = lax.dot_general(a, w_ref[...], (((1,), (1,)), ((), ())),
                        preferred_element_type=jnp.float32)
    o_ref[...] = jnp.maximum(h + b_ref[...], 0.0)


def _tc_linear(agg2, W, b):
    return pl.pallas_call(
        _tc_linear_body,
        grid=(N_PAD // _BLK,),
        in_specs=[
            pl.BlockSpec((NC, _BLK, D), lambda i: (0, i, 0)),
            pl.BlockSpec((D, D), lambda i: (0, 0)),
            pl.BlockSpec((1, D), lambda i: (0, 0)),
        ],
        out_specs=pl.BlockSpec((_BLK, D), lambda i: (i, 0)),
        out_shape=jax.ShapeDtypeStruct((N_PAD, D), jnp.float32),
    )(agg2, W, b.reshape(1, D))


def kernel(x, edge_index, W, b):
    ei = edge_index.astype(jnp.int32)
    src = ei[0].reshape(NW, NITER, CHUNK)
    dst = ei[1].reshape(NW, NITER, CHUNK)
    zeros = jnp.zeros((STRIPE, D), jnp.float32)
    agg2 = _sc_aggregate(x, src, dst, zeros)
    return _tc_linear(agg2, W, b)[:N]
